# baseline (speedup 1.0000x reference)
import functools

import jax
import jax.numpy as jnp
from jax import lax
from jax.experimental import pallas as pl
from jax.experimental.pallas import tpu as pltpu
from jax.experimental.pallas import tpu_sc as plsc

_N = 2048
_SLABS = 16
_ROWS = _N // _SLABS
_RB = 8
_NW = 32
_TPW = 20 * _SLABS // _NW
_LANES = 16
_CHUNKS = _N // _LANES


def _sc_body(qr_hbm, q2_hbm, cr_hbm, c2_hbm, out_hbm,
             cxa, cxb, cxc, c2b, qxa, qxb, qxc, q2b, acc):
    wid = lax.axis_index("s") * 2 + lax.axis_index("c")
    for k in range(_TPW):
        t = wid * _TPW + k
        q = t // _SLABS
        slab = t % _SLABS
        cbase = q * (3 * _N)
        pltpu.sync_copy(cr_hbm.at[pl.ds(cbase, _N)], cxa)
        pltpu.sync_copy(cr_hbm.at[pl.ds(cbase + _N, _N)], cxb)
        pltpu.sync_copy(cr_hbm.at[pl.ds(cbase + 2 * _N, _N)], cxc)
        pltpu.sync_copy(c2_hbm.at[pl.ds(q * _N, _N)], c2b)
        qbase = q * (3 * _N) + slab * _ROWS
        pltpu.sync_copy(qr_hbm.at[pl.ds(qbase, _ROWS)],
                        qxa.at[pl.ds(0, _ROWS)])
        pltpu.sync_copy(qr_hbm.at[pl.ds(qbase + _N, _ROWS)],
                        qxb.at[pl.ds(0, _ROWS)])
        pltpu.sync_copy(qr_hbm.at[pl.ds(qbase + 2 * _N, _ROWS)],
                        qxc.at[pl.ds(0, _ROWS)])
        pltpu.sync_copy(q2_hbm.at[pl.ds(q * _N + slab * _ROWS, _ROWS)],
                        q2b.at[pl.ds(0, _ROWS)])

        def rb_body(rb, acc_s):
            va = qxa[pl.ds(rb * _RB, _LANES)]
            vb = qxb[pl.ds(rb * _RB, _LANES)]
            vc = qxc[pl.ds(rb * _RB, _LANES)]
            v2 = q2b[pl.ds(rb * _RB, _LANES)]
            b0 = [jnp.full((_LANES,), va[j] * -2.0, jnp.float32)
                  for j in range(_RB)]
            b1 = [jnp.full((_LANES,), vb[j] * -2.0, jnp.float32)
                  for j in range(_RB)]
            b2 = [jnp.full((_LANES,), vc[j] * -2.0, jnp.float32)
                  for j in range(_RB)]

            def mc_body(mc, rms):
                ya = cxa[pl.ds(mc * _LANES, _LANES)]
                yb = cxb[pl.ds(mc * _LANES, _LANES)]
                yc = cxc[pl.ds(mc * _LANES, _LANES)]
                y2 = c2b[pl.ds(mc * _LANES, _LANES)]
                out = []
                for j in range(_RB):
                    p = b0[j] * ya + b1[j] * yb + b2[j] * yc
                    d = y2 + p
                    out.append(jnp.minimum(rms[j], d))
                return tuple(out)

            init = tuple(
                jnp.full((_LANES,), jnp.inf, jnp.float32) for _ in range(_RB))
            rms = lax.fori_loop(0, _CHUNKS, mc_body, init, unroll=False)
            for j in range(_RB):
                acc_s = acc_s + jnp.maximum(jnp.min(rms[j]) + v2[j], 0.0)
            return acc_s

        total = lax.fori_loop(0, _ROWS // _RB, rb_body, jnp.float32(0.0),
                              unroll=False)
        acc[pl.ds(k * _LANES, _LANES)] = jnp.full((_LANES,), total,
                                                  jnp.float32)
    pltpu.sync_copy(acc, out_hbm.at[pl.ds(wid * (_TPW * _LANES),
                                          _TPW * _LANES)])


@jax.jit
def _chamfer_sc(qr, q2, cr, c2):
    mesh = plsc.VectorSubcoreMesh(core_axis_name="c", subcore_axis_name="s")
    fn = pl.kernel(
        _sc_body,
        mesh=mesh,
        compiler_params=pltpu.CompilerParams(needs_layout_passes=False),
        out_type=jax.ShapeDtypeStruct((_NW * _TPW * _LANES,), jnp.float32),
        scratch_types=[
            pltpu.VMEM((_N,), jnp.float32),
            pltpu.VMEM((_N,), jnp.float32),
            pltpu.VMEM((_N,), jnp.float32),
            pltpu.VMEM((_N,), jnp.float32),
            pltpu.VMEM((_ROWS + _LANES,), jnp.float32),
            pltpu.VMEM((_ROWS + _LANES,), jnp.float32),
            pltpu.VMEM((_ROWS + _LANES,), jnp.float32),
            pltpu.VMEM((_ROWS + _LANES,), jnp.float32),
            pltpu.VMEM((_TPW * _LANES,), jnp.float32),
        ],
    )
    return fn(qr, q2, cr, c2)


def kernel(output_points, target_points, n_samples):
    b, s, n, _ = output_points.shape
    p = b * s
    x = output_points.reshape(p, n, 3)
    y = target_points.reshape(p, n, 3)
    qpts = jnp.concatenate([x, y], axis=0)
    cpts = jnp.concatenate([y, x], axis=0)
    q2 = jnp.sum(qpts * qpts, axis=-1).reshape(-1)
    c2 = jnp.sum(cpts * cpts, axis=-1).reshape(-1)
    qr = qpts.astype(jnp.bfloat16).astype(jnp.float32)
    cr = cpts.astype(jnp.bfloat16).astype(jnp.float32)
    qr = qr.transpose(0, 2, 1).reshape(-1)
    cr = cr.transpose(0, 2, 1).reshape(-1)
    raw = _chamfer_sc(qr, q2, cr, c2)
    partial = raw.reshape(_NW * _TPW, _LANES)[:, 0]
    d_mean = partial.reshape(2 * p, _SLABS).sum(axis=1) / n
    per_pair = (d_mean[:p] + d_mean[p:]).reshape(b, s)
    tensor = per_pair.T
    means = jnp.mean(tensor, axis=1)
    return (means, tensor)


# speedup vs baseline: 4.3673x; 4.3673x over previous
import jax
import jax.numpy as jnp
from jax import lax
from jax.experimental import pallas as pl
from jax.experimental.pallas import tpu as pltpu
from jax.experimental.pallas import tpu_sc as plsc

_N = 2048
_TILE = 256
_RB = 8
_NW = 32
_LANES = 16
_CHUNKS = _N // _LANES

_KSC = 0


def _tc_body(q_ref, c_ref, q2_ref, c2_ref, out_ref):
    qa = q_ref[0]
    ca = c_ref[0]
    q2v = q2_ref[0][0]
    c2v = c2_ref[0][0]
    total = jnp.float32(0.0)
    for t in range(_N // _TILE):
        qt = qa[:, t * _TILE:(t + 1) * _TILE]
        xyp = jax.lax.dot_general(
            qt, ca, (((0,), (0,)), ((), ())),
            preferred_element_type=jnp.float32)
        rm = jnp.min(xyp + c2v[None, :], axis=1)
        q2t = q2v[t * _TILE:(t + 1) * _TILE]
        total += jnp.sum(jnp.maximum(rm + q2t, 0.0))
    out_ref[pl.program_id(0), 0] = total


def _tc_dir(q, c, q2, c2):
    p = q.shape[0]
    return pl.pallas_call(
        _tc_body,
        grid=(p,),
        in_specs=[
            pl.BlockSpec((1, 8, _N), lambda i: (i, 0, 0)),
            pl.BlockSpec((1, 8, _N), lambda i: (i, 0, 0)),
            pl.BlockSpec((1, 1, _N), lambda i: (i, 0, 0)),
            pl.BlockSpec((1, 1, _N), lambda i: (i, 0, 0)),
        ],
        out_specs=pl.BlockSpec((p, 1), lambda i: (0, 0),
                               memory_space=pltpu.SMEM),
        out_shape=jax.ShapeDtypeStruct((p, 1), jnp.float32),
    )(q, c, q2, c2)


def _bf(v):
    return v.astype(jnp.bfloat16).astype(jnp.float32)


def _tc_prep(qpts, cpts):
    p, n, _ = qpts.shape
    qc = qpts.transpose(0, 2, 1)
    cc = cpts.transpose(0, 2, 1)
    zeros = jnp.zeros((p, 5, n), jnp.float32)
    q = jnp.concatenate([-2.0 * qc, zeros], axis=1)
    c = jnp.concatenate([cc, zeros], axis=1)
    q2 = jnp.sum(qpts * qpts, axis=-1)[:, None]
    c2 = jnp.sum(cpts * cpts, axis=-1)[:, None]
    return q, c, q2, c2


_RPT = _KSC * _N // _NW if _KSC else 0


def _sc_body(qr_hbm, q2_hbm, cr_hbm, c2_hbm, out_hbm,
             cxa, cxb, cxc, c2b, qxa, qxb, qxc, q2b, acc):
    wid = lax.axis_index("s") * 2 + lax.axis_index("c")
    g0 = wid * _RPT
    q = g0 // _N
    row0 = g0 - q * _N
    cbase = q * (3 * _N)
    pltpu.sync_copy(cr_hbm.at[pl.ds(cbase, _N)], cxa)
    pltpu.sync_copy(cr_hbm.at[pl.ds(cbase + _N, _N)], cxb)
    pltpu.sync_copy(cr_hbm.at[pl.ds(cbase + 2 * _N, _N)], cxc)
    pltpu.sync_copy(c2_hbm.at[pl.ds(q * _N, _N)], c2b)
    qbase = cbase + row0
    pltpu.sync_copy(qr_hbm.at[pl.ds(qbase, _RPT)], qxa.at[pl.ds(0, _RPT)])
    pltpu.sync_copy(qr_hbm.at[pl.ds(qbase + _N, _RPT)],
                    qxb.at[pl.ds(0, _RPT)])
    pltpu.sync_copy(qr_hbm.at[pl.ds(qbase + 2 * _N, _RPT)],
                    qxc.at[pl.ds(0, _RPT)])
    pltpu.sync_copy(q2_hbm.at[pl.ds(q * _N + row0, _RPT)],
                    q2b.at[pl.ds(0, _RPT)])

    def rb_body(rb, acc_s):
        va = qxa[pl.ds(rb * _RB, _LANES)]
        vb = qxb[pl.ds(rb * _RB, _LANES)]
        vc = qxc[pl.ds(rb * _RB, _LANES)]
        v2 = q2b[pl.ds(rb * _RB, _LANES)]
        b0 = [jnp.full((_LANES,), va[j] * -2.0, jnp.float32)
              for j in range(_RB)]
        b1 = [jnp.full((_LANES,), vb[j] * -2.0, jnp.float32)
              for j in range(_RB)]
        b2 = [jnp.full((_LANES,), vc[j] * -2.0, jnp.float32)
              for j in range(_RB)]

        def mc_body(mc, rms):
            ya = cxa[pl.ds(mc * _LANES, _LANES)]
            yb = cxb[pl.ds(mc * _LANES, _LANES)]
            yc = cxc[pl.ds(mc * _LANES, _LANES)]
            y2 = c2b[pl.ds(mc * _LANES, _LANES)]
            out = []
            for j in range(_RB):
                p = b0[j] * ya + b1[j] * yb + b2[j] * yc
                d = y2 + p
                out.append(jnp.minimum(rms[j], d))
            return tuple(out)

        init = tuple(
            jnp.full((_LANES,), jnp.inf, jnp.float32) for _ in range(_RB))
        rms = lax.fori_loop(0, _CHUNKS, mc_body, init, unroll=False)
        for j in range(_RB):
            acc_s = acc_s + jnp.maximum(jnp.min(rms[j]) + v2[j], 0.0)
        return acc_s

    total = lax.fori_loop(0, _RPT // _RB, rb_body, jnp.float32(0.0),
                          unroll=False)
    acc[...] = jnp.full((_LANES,), total, jnp.float32)
    pltpu.sync_copy(acc, out_hbm.at[pl.ds(wid * _LANES, _LANES)])


def _sc_dir(qr, q2, cr, c2):
    mesh = plsc.VectorSubcoreMesh(core_axis_name="c", subcore_axis_name="s")
    fn = pl.kernel(
        _sc_body,
        mesh=mesh,
        compiler_params=pltpu.CompilerParams(needs_layout_passes=False),
        out_type=jax.ShapeDtypeStruct((_NW * _LANES,), jnp.float32),
        scratch_types=[
            pltpu.VMEM((_N,), jnp.float32),
            pltpu.VMEM((_N,), jnp.float32),
            pltpu.VMEM((_N,), jnp.float32),
            pltpu.VMEM((_N,), jnp.float32),
            pltpu.VMEM((_RPT + _LANES,), jnp.float32),
            pltpu.VMEM((_RPT + _LANES,), jnp.float32),
            pltpu.VMEM((_RPT + _LANES,), jnp.float32),
            pltpu.VMEM((_RPT + _LANES,), jnp.float32),
            pltpu.VMEM((_LANES,), jnp.float32),
        ],
    )
    return fn(qr, q2, cr, c2)


@jax.jit
def _chamfer(qpts, cpts):
    twop = qpts.shape[0]
    if _KSC > 0:
        qs, cs = qpts[:_KSC], cpts[:_KSC]
        q2s = jnp.sum(qs * qs, axis=-1).reshape(-1)
        c2s = jnp.sum(cs * cs, axis=-1).reshape(-1)
        qrs = _bf(qs).transpose(0, 2, 1).reshape(-1)
        crs = _bf(cs).transpose(0, 2, 1).reshape(-1)
        raw = _sc_dir(qrs, q2s, crs, c2s)
        per_worker = raw.reshape(_NW, _LANES)[:, 0]
        sc_sums = per_worker.reshape(_KSC, _NW // _KSC).sum(axis=1)
    if _KSC < twop:
        q, c, q2, c2 = _tc_prep(qpts[_KSC:], cpts[_KSC:])
        tc_sums = _tc_dir(q, c, q2, c2)[:, 0]
    if _KSC == 0:
        return tc_sums
    if _KSC == twop:
        return sc_sums
    return jnp.concatenate([sc_sums, tc_sums])


def kernel(output_points, target_points, n_samples):
    b, s, n, _ = output_points.shape
    p = b * s
    x = output_points.reshape(p, n, 3)
    y = target_points.reshape(p, n, 3)
    qpts = jnp.concatenate([x, y], axis=0)
    cpts = jnp.concatenate([y, x], axis=0)
    sums = _chamfer(qpts, cpts)
    d_mean = sums / n
    per_pair = (d_mean[:p] + d_mean[p:]).reshape(b, s)
    tensor = per_pair.T
    means = jnp.mean(tensor, axis=1)
    return (means, tensor)


# speedup vs baseline: 5.8133x; 1.3311x over previous
import jax
import jax.numpy as jnp
from jax import lax
from jax.experimental import pallas as pl
from jax.experimental.pallas import tpu as pltpu
from jax.experimental.pallas import tpu_sc as plsc

_N = 2048
_TILE = 256
_RB = 8
_NW = 32
_LANES = 16
_CHUNKS = _N // _LANES

_KSC = 0


def _tc_body(q_ref, c_ref, q2_ref, c2_ref, out_ref):
    qa = q_ref[0]
    ca = c_ref[0]
    q2v = q2_ref[0][0]
    c2v = c2_ref[0][0]
    total = jnp.float32(0.0)
    colmin = jnp.full((_N,), jnp.inf, jnp.float32)
    for t in range(_N // _TILE):
        qt = qa[:, t * _TILE:(t + 1) * _TILE]
        q2t = q2v[t * _TILE:(t + 1) * _TILE]
        xyp = jax.lax.dot_general(
            qt, ca, (((0,), (0,)), ((), ())),
            preferred_element_type=jnp.float32)
        rm = jnp.min(xyp + c2v[None, :], axis=1)
        total += jnp.sum(jnp.maximum(rm + q2t, 0.0))
        colmin = jnp.minimum(colmin, jnp.min(xyp + q2t[:, None], axis=0))
    total += jnp.sum(jnp.maximum(colmin + c2v, 0.0))
    out_ref[pl.program_id(0), 0] = total


def _tc_dir(q, c, q2, c2):
    p = q.shape[0]
    return pl.pallas_call(
        _tc_body,
        grid=(p,),
        in_specs=[
            pl.BlockSpec((1, 8, _N), lambda i: (i, 0, 0)),
            pl.BlockSpec((1, 8, _N), lambda i: (i, 0, 0)),
            pl.BlockSpec((1, 1, _N), lambda i: (i, 0, 0)),
            pl.BlockSpec((1, 1, _N), lambda i: (i, 0, 0)),
        ],
        out_specs=pl.BlockSpec((p, 1), lambda i: (0, 0),
                               memory_space=pltpu.SMEM),
        out_shape=jax.ShapeDtypeStruct((p, 1), jnp.float32),
    )(q, c, q2, c2)


def _bf(v):
    return v.astype(jnp.bfloat16).astype(jnp.float32)


def _tc_prep(qpts, cpts):
    p, n, _ = qpts.shape
    qc = qpts.transpose(0, 2, 1)
    cc = cpts.transpose(0, 2, 1)
    zeros = jnp.zeros((p, 5, n), jnp.float32)
    q = jnp.concatenate([-2.0 * qc, zeros], axis=1)
    c = jnp.concatenate([cc, zeros], axis=1)
    q2 = jnp.sum(qpts * qpts, axis=-1)[:, None]
    c2 = jnp.sum(cpts * cpts, axis=-1)[:, None]
    return q, c, q2, c2


_RPT = 2 * _KSC * _N // _NW if _KSC else 0


def _sc_body(qr_hbm, q2_hbm, cr_hbm, c2_hbm, out_hbm,
             cxa, cxb, cxc, c2b, qxa, qxb, qxc, q2b, acc):
    wid = lax.axis_index("s") * 2 + lax.axis_index("c")
    g0 = wid * _RPT
    q = g0 // _N
    row0 = g0 - q * _N
    cbase = q * (3 * _N)
    pltpu.sync_copy(cr_hbm.at[pl.ds(cbase, _N)], cxa)
    pltpu.sync_copy(cr_hbm.at[pl.ds(cbase + _N, _N)], cxb)
    pltpu.sync_copy(cr_hbm.at[pl.ds(cbase + 2 * _N, _N)], cxc)
    pltpu.sync_copy(c2_hbm.at[pl.ds(q * _N, _N)], c2b)
    qbase = cbase + row0
    pltpu.sync_copy(qr_hbm.at[pl.ds(qbase, _RPT)], qxa.at[pl.ds(0, _RPT)])
    pltpu.sync_copy(qr_hbm.at[pl.ds(qbase + _N, _RPT)],
                    qxb.at[pl.ds(0, _RPT)])
    pltpu.sync_copy(qr_hbm.at[pl.ds(qbase + 2 * _N, _RPT)],
                    qxc.at[pl.ds(0, _RPT)])
    pltpu.sync_copy(q2_hbm.at[pl.ds(q * _N + row0, _RPT)],
                    q2b.at[pl.ds(0, _RPT)])

    def rb_body(rb, acc_s):
        va = qxa[pl.ds(rb * _RB, _LANES)]
        vb = qxb[pl.ds(rb * _RB, _LANES)]
        vc = qxc[pl.ds(rb * _RB, _LANES)]
        v2 = q2b[pl.ds(rb * _RB, _LANES)]
        b0 = [jnp.full((_LANES,), va[j] * -2.0, jnp.float32)
              for j in range(_RB)]
        b1 = [jnp.full((_LANES,), vb[j] * -2.0, jnp.float32)
              for j in range(_RB)]
        b2 = [jnp.full((_LANES,), vc[j] * -2.0, jnp.float32)
              for j in range(_RB)]

        def mc_body(mc, rms):
            ya = cxa[pl.ds(mc * _LANES, _LANES)]
            yb = cxb[pl.ds(mc * _LANES, _LANES)]
            yc = cxc[pl.ds(mc * _LANES, _LANES)]
            y2 = c2b[pl.ds(mc * _LANES, _LANES)]
            out = []
            for j in range(_RB):
                p = b0[j] * ya + b1[j] * yb + b2[j] * yc
                d = y2 + p
                out.append(jnp.minimum(rms[j], d))
            return tuple(out)

        init = tuple(
            jnp.full((_LANES,), jnp.inf, jnp.float32) for _ in range(_RB))
        rms = lax.fori_loop(0, _CHUNKS, mc_body, init, unroll=False)
        for j in range(_RB):
            acc_s = acc_s + jnp.maximum(jnp.min(rms[j]) + v2[j], 0.0)
        return acc_s

    total = lax.fori_loop(0, _RPT // _RB, rb_body, jnp.float32(0.0),
                          unroll=False)
    acc[...] = jnp.full((_LANES,), total, jnp.float32)
    pltpu.sync_copy(acc, out_hbm.at[pl.ds(wid * _LANES, _LANES)])


def _sc_dir(qr, q2, cr, c2):
    mesh = plsc.VectorSubcoreMesh(core_axis_name="c", subcore_axis_name="s")
    fn = pl.kernel(
        _sc_body,
        mesh=mesh,
        compiler_params=pltpu.CompilerParams(needs_layout_passes=False),
        out_type=jax.ShapeDtypeStruct((_NW * _LANES,), jnp.float32),
        scratch_types=[
            pltpu.VMEM((_N,), jnp.float32),
            pltpu.VMEM((_N,), jnp.float32),
            pltpu.VMEM((_N,), jnp.float32),
            pltpu.VMEM((_N,), jnp.float32),
            pltpu.VMEM((_RPT + _LANES,), jnp.float32),
            pltpu.VMEM((_RPT + _LANES,), jnp.float32),
            pltpu.VMEM((_RPT + _LANES,), jnp.float32),
            pltpu.VMEM((_RPT + _LANES,), jnp.float32),
            pltpu.VMEM((_LANES,), jnp.float32),
        ],
    )
    return fn(qr, q2, cr, c2)


@jax.jit
def _chamfer(x, y):
    p = x.shape[0]
    if _KSC > 0:
        qs = jnp.concatenate([x[:_KSC], y[:_KSC]], axis=0)
        cs = jnp.concatenate([y[:_KSC], x[:_KSC]], axis=0)
        q2s = jnp.sum(qs * qs, axis=-1).reshape(-1)
        c2s = jnp.sum(cs * cs, axis=-1).reshape(-1)
        qrs = _bf(qs).transpose(0, 2, 1).reshape(-1)
        crs = _bf(cs).transpose(0, 2, 1).reshape(-1)
        raw = _sc_dir(qrs, q2s, crs, c2s)
        per_worker = raw.reshape(_NW, _LANES)[:, 0]
        dir_sums = per_worker.reshape(2 * _KSC, _NW // (2 * _KSC)).sum(axis=1)
        sc_sums = dir_sums[:_KSC] + dir_sums[_KSC:]
    if _KSC < p:
        q, c, q2, c2 = _tc_prep(x[_KSC:], y[_KSC:])
        tc_sums = _tc_dir(q, c, q2, c2)[:, 0]
    if _KSC == 0:
        return tc_sums
    if _KSC == p:
        return sc_sums
    return jnp.concatenate([sc_sums, tc_sums])


def kernel(output_points, target_points, n_samples):
    b, s, n, _ = output_points.shape
    p = b * s
    x = output_points.reshape(p, n, 3)
    y = target_points.reshape(p, n, 3)
    per_pair = (_chamfer(x, y) / n).reshape(b, s)
    tensor = per_pair.T
    means = jnp.mean(tensor, axis=1)
    return (means, tensor)
